# baseline (speedup 1.0000x reference)
import functools

import jax
import jax.numpy as jnp
from jax import lax
from jax.experimental import pallas as pl
from jax.experimental.pallas import tpu as pltpu
from jax.experimental.pallas import tpu_sc as plsc

B, L, D = 4096, 200, 64
ROWS = L * D
NC, NS = 2, 16
NW = NC * NS
RPW = ROWS // NW
CB = 8
NCH = RPW // CB
LANES = 16
SPAD = RPW + LANES

_mesh = plsc.VectorSubcoreMesh(core_axis_name="c", subcore_axis_name="s")


@functools.partial(
    pl.kernel,
    out_type=jax.ShapeDtypeStruct((ROWS, B), jnp.float32),
    mesh=_mesh,
    scratch_types=[
        pltpu.VMEM((SPAD,), jnp.float32),
        pltpu.VMEM((3, CB, B), jnp.float32),
        pltpu.SemaphoreType.DMA,
        pltpu.SemaphoreType.DMA,
        pltpu.SemaphoreType.DMA,
    ],
)
def _broadcast_table(emb_hbm, out_hbm, scal, fbuf, sem0, sem1, sem2):
    wid = lax.axis_index("s") * NC + lax.axis_index("c")
    row0 = wid * RPW
    pltpu.sync_copy(emb_hbm.at[pl.ds(row0, RPW)], scal.at[pl.ds(0, RPW)])
    sems = (sem0, sem1, sem2)

    def fill_row(buf_idx, r, ridx):
        vchunk = scal[pl.ds(ridx, LANES)]
        vec = jnp.full((LANES,), vchunk[0], jnp.float32)

        def body(k, carry):
            base = k * (8 * LANES)
            for u in range(8):
                fbuf[buf_idx, r, pl.ds(base + u * LANES, LANES)] = vec
            return carry

        lax.fori_loop(0, B // (8 * LANES), body, 0)

    def do_chunk(i, b, c, first):
        dst = out_hbm.at[pl.ds(row0 + c * CB, CB)]

        @pl.when(i > 0 if first else jnp.bool_(True))
        def _wait():
            pltpu.make_async_copy(fbuf.at[b], dst, sems[b]).wait()

        for r in range(CB):
            fill_row(b, r, c * CB + r)
        pltpu.async_copy(fbuf.at[b], dst, sems[b])

    def chunk(i, carry):
        for b in range(3):
            do_chunk(i, b, i * 3 + b, True)
        return carry

    NLOOP = NCH // 3
    lax.fori_loop(0, NLOOP, chunk, 0)
    for b in range(NCH - NLOOP * 3):
        do_chunk(1, b, NLOOP * 3 + b, False)
    for b in range(3):
        pltpu.make_async_copy(
            fbuf.at[b], out_hbm.at[pl.ds(row0, CB)], sems[b]
        ).wait()


def kernel(x, embedding):
    img = _broadcast_table(jnp.reshape(embedding, (ROWS,)))
    return jnp.transpose(jnp.reshape(img, (L, D, B)), (2, 0, 1))


# speedup vs baseline: 1.0185x; 1.0185x over previous
import functools

import jax
import jax.numpy as jnp
from jax import lax
from jax.experimental import pallas as pl
from jax.experimental.pallas import tpu as pltpu
from jax.experimental.pallas import tpu_sc as plsc

B, L, D = 4096, 200, 64
ROWS = L * D
NC, NS = 2, 16
NW = NC * NS
RPW = ROWS // NW
CB = 8
NCH = RPW // CB
LANES = 16
SPAD = RPW + LANES

_mesh = plsc.VectorSubcoreMesh(core_axis_name="c", subcore_axis_name="s")


@functools.partial(
    pl.kernel,
    out_type=jax.ShapeDtypeStruct((ROWS, B), jnp.float32),
    mesh=_mesh,
    scratch_types=[
        pltpu.VMEM((SPAD,), jnp.float32),
        pltpu.VMEM((2, CB, B), jnp.float32),
        pltpu.SemaphoreType.DMA,
        pltpu.SemaphoreType.DMA,
    ],
)
def _broadcast_table(emb_hbm, out_hbm, scal, fbuf, sem0, sem1):
    wid = lax.axis_index("s") * NC + lax.axis_index("c")
    row0 = wid * RPW
    pltpu.sync_copy(emb_hbm.at[pl.ds(row0, RPW)], scal.at[pl.ds(0, RPW)])
    sems = (sem0, sem1)
    lane = lax.iota(jnp.int32, LANES)

    def fill_row(buf_idx, r, ridx):
        vchunk = scal[pl.ds(ridx, LANES)]
        vec = jnp.full((LANES,), vchunk[0], jnp.float32)

        def body(k, carry):
            base = k * (16 * LANES)
            for u in range(16):
                fbuf[buf_idx, r, pl.ds(base + u * LANES, LANES)] = vec
            return carry

        lax.fori_loop(0, B // (16 * LANES), body, 0)

    def chunk(i, carry):
        for b in range(2):
            c = i * 2 + b
            dst = out_hbm.at[pl.ds(row0 + c * CB, CB)]

            @pl.when(i > 0)
            def _wait():
                pltpu.make_async_copy(fbuf.at[b], dst, sems[b]).wait()

            for r in range(CB):
                fill_row(b, r, c * CB + r)
            pltpu.async_copy(fbuf.at[b], dst, sems[b])
        return carry

    lax.fori_loop(0, NCH // 2, chunk, 0)
    for b in range(2):
        pltpu.make_async_copy(
            fbuf.at[b], out_hbm.at[pl.ds(row0, CB)], sems[b]
        ).wait()


def kernel(x, embedding):
    img = _broadcast_table(jnp.reshape(embedding, (ROWS,)))
    return jnp.transpose(jnp.reshape(img, (L, D, B)), (2, 0, 1))


# speedup vs baseline: 1.0290x; 1.0103x over previous
import functools

import jax
import jax.numpy as jnp
from jax import lax
from jax.experimental import pallas as pl
from jax.experimental.pallas import tpu as pltpu
from jax.experimental.pallas import tpu_sc as plsc

B, L, D = 4096, 200, 64
ROWS = L * D
NC, NS = 2, 16
NW = NC * NS
RPW = ROWS // NW
CB = 8
NCH = RPW // CB
LANES = 16
SPAD = RPW + LANES

_mesh = plsc.VectorSubcoreMesh(core_axis_name="c", subcore_axis_name="s")


@functools.partial(
    pl.kernel,
    out_type=jax.ShapeDtypeStruct((ROWS, B), jnp.float32),
    mesh=_mesh,
    scratch_types=[
        pltpu.VMEM((SPAD,), jnp.float32),
        pltpu.VMEM((2, CB, B), jnp.float32),
        pltpu.SemaphoreType.DMA,
        pltpu.SemaphoreType.DMA,
    ],
)
def _broadcast_table(emb_hbm, out_hbm, scal, fbuf, sem0, sem1):
    wid = lax.axis_index("s") * NC + lax.axis_index("c")
    row0 = wid * RPW
    pltpu.sync_copy(emb_hbm.at[pl.ds(row0, RPW)], scal.at[pl.ds(0, RPW)])
    sems = (sem0, sem1)

    def fill_row(buf_idx, r, ridx):
        vchunk = scal[pl.ds(ridx, LANES)]
        vec = jnp.full((LANES,), vchunk[0], jnp.float32)

        def body(k, carry):
            base = k * (8 * LANES)
            for u in range(8):
                fbuf[buf_idx, r, pl.ds(base + u * LANES, LANES)] = vec
            return carry

        lax.fori_loop(0, B // (8 * LANES), body, 0)

    def chunk(i, carry):
        for b in range(2):
            c = i * 2 + b
            dst = out_hbm.at[pl.ds(row0 + c * CB, CB)]

            @pl.when(i > 0)
            def _wait():
                pltpu.make_async_copy(fbuf.at[b], dst, sems[b]).wait()

            for r in range(CB):
                fill_row(b, r, c * CB + r)
            pltpu.async_copy(fbuf.at[b], dst, sems[b])
        return carry

    lax.fori_loop(0, NCH // 2, chunk, 0)
    for b in range(2):
        pltpu.make_async_copy(
            fbuf.at[b], out_hbm.at[pl.ds(row0, CB)], sems[b]
        ).wait()


def kernel(x, embedding):
    img = _broadcast_table(jnp.reshape(embedding, (ROWS,)))
    return jnp.transpose(jnp.reshape(img, (L, D, B)), (2, 0, 1))
